# baseline (speedup 1.0000x reference)
import jax
import jax.numpy as jnp
from jax.experimental import pallas as pl
from jax.experimental.pallas import tpu as pltpu

_T = 2048


def _linoss_body(x_ref, br_ref, bi_ref, cr_ref, ci_ref, d_ref, ad_ref,
                 st_ref, o_ref, carry_ref, cosT_ref, sinT_ref, h0_ref,
                 h1_ref, p2_ref, btr_ref, bti_ref, ctr_ref, cti_ref):
    i = pl.program_id(0)
    T = x_ref.shape[0]
    P = ad_ref.shape[1]

    s = jax.nn.sigmoid(st_ref[...])

    @pl.when(i == 0)
    def _():
        carry_ref[...] = jnp.zeros_like(carry_ref)
        btr_ref[...] = br_ref[...].T * s
        bti_ref[...] = bi_ref[...].T * s
        ctr_ref[...] = cr_ref[...].T
        cti_ref[...] = ci_ref[...].T
        a = jnp.maximum(ad_ref[...], 0.0)
        e2 = s * s * a
        cth = 1.0 - 0.5 * e2
        sth = jnp.sqrt(jnp.maximum(e2 * (1.0 - 0.25 * e2), 1e-24))
        u3 = jax.lax.broadcasted_iota(jnp.int32, (T, 1), 0)
        cosT = jnp.ones((T, P), jnp.float32)
        sinT = jnp.zeros((T, P), jnp.float32)
        ck, sk = cth, sth
        d = 1
        while d < T:
            z = jnp.zeros((d, P), jnp.float32)
            shc = jnp.concatenate([z, cosT[:T - d]], axis=0)
            shs = jnp.concatenate([z, sinT[:T - d]], axis=0)
            sel = u3 >= d
            cosT = jnp.where(sel, shc * ck - shs * sk, cosT)
            sinT = jnp.where(sel, shs * ck + shc * sk, sinT)
            ck, sk = ck * ck - sk * sk, 2.0 * sk * ck
            d *= 2
        cosT_ref[...] = cosT
        sinT_ref[...] = sinT
        isth = 1.0 / sth
        cot = cth * isth
        h0 = sinT * cot + cosT
        p2 = cosT * cot - sinT
        h0_ref[...] = h0
        h1_ref[...] = h0 * cth + p2 * sth
        p2_ref[...] = p2

    x = x_ref[...]
    f_r = jnp.dot(x, btr_ref[...], preferred_element_type=jnp.float32)
    f_i = jnp.dot(x, bti_ref[...], preferred_element_type=jnp.float32)

    cc = carry_ref[...]
    zm1r, zm1i = cc[0:1], cc[1:2]
    zm2r, zm2i = cc[2:3], cc[3:4]
    fpr, fpi = cc[4:5], cc[5:6]

    sm1 = s - 1.0
    fshr = jnp.concatenate([fpr, f_r[:T - 1]], axis=0)
    fshi = jnp.concatenate([fpi, f_i[:T - 1]], axis=0)
    g_r = f_r + sm1 * fshr
    g_i = f_i + sm1 * fshi

    cosT = cosT_ref[...]
    sinT = sinT_ref[...]
    qcr = cosT * g_r
    qci = cosT * g_i
    qsr = sinT * g_r
    qsi = sinT * g_i

    d = 1
    while d < T:
        z = jnp.zeros((d, P), jnp.float32)
        qcr = qcr + jnp.concatenate([z, qcr[:T - d]], axis=0)
        qci = qci + jnp.concatenate([z, qci[:T - d]], axis=0)
        qsr = qsr + jnp.concatenate([z, qsr[:T - d]], axis=0)
        qsi = qsi + jnp.concatenate([z, qsi[:T - d]], axis=0)
        d *= 2

    h0 = h0_ref[...]
    h1 = h1_ref[...]
    p2 = p2_ref[...]
    z_r = h0 * qcr - p2 * qsr + zm1r * h1 - zm2r * h0
    z_i = h0 * qci - p2 * qsi + zm1i * h1 - zm2i * h0

    carry_ref[0:1] = z_r[T - 1:T]
    carry_ref[1:2] = z_i[T - 1:T]
    carry_ref[2:3] = z_r[T - 2:T - 1]
    carry_ref[3:4] = z_i[T - 2:T - 1]
    carry_ref[4:5] = f_r[T - 1:T]
    carry_ref[5:6] = f_i[T - 1:T]

    o = (jnp.dot(z_r, ctr_ref[...], preferred_element_type=jnp.float32)
         - jnp.dot(z_i, cti_ref[...], preferred_element_type=jnp.float32)
         + x * d_ref[...])
    o_ref[...] = o


def kernel(input_sequence, A_diag_raw, B_real, B_img, C_real, C_img, D,
           steps_raw):
    L, H = input_sequence.shape
    P = A_diag_raw.shape[0]
    n_chunks = L // _T

    return pl.pallas_call(
        _linoss_body,
        out_shape=jax.ShapeDtypeStruct((L, H), jnp.float32),
        grid=(n_chunks,),
        in_specs=[
            pl.BlockSpec((_T, H), lambda i: (i, 0)),
            pl.BlockSpec((P, H), lambda i: (0, 0)),
            pl.BlockSpec((P, H), lambda i: (0, 0)),
            pl.BlockSpec((H, P), lambda i: (0, 0)),
            pl.BlockSpec((H, P), lambda i: (0, 0)),
            pl.BlockSpec((1, H), lambda i: (0, 0)),
            pl.BlockSpec((1, P), lambda i: (0, 0)),
            pl.BlockSpec((1, P), lambda i: (0, 0)),
        ],
        out_specs=pl.BlockSpec((_T, H), lambda i: (i, 0)),
        scratch_shapes=[
            pltpu.VMEM((8, P), jnp.float32),
            pltpu.VMEM((_T, P), jnp.float32),
            pltpu.VMEM((_T, P), jnp.float32),
            pltpu.VMEM((_T, P), jnp.float32),
            pltpu.VMEM((_T, P), jnp.float32),
            pltpu.VMEM((_T, P), jnp.float32),
            pltpu.VMEM((H, P), jnp.float32),
            pltpu.VMEM((H, P), jnp.float32),
            pltpu.VMEM((P, H), jnp.float32),
            pltpu.VMEM((P, H), jnp.float32),
        ],
        compiler_params=pltpu.CompilerParams(
            dimension_semantics=("arbitrary",),
        ),
        name="linoss_scan",
    )(
        input_sequence,
        B_real, B_img,
        C_real, C_img,
        D.reshape(1, H),
        A_diag_raw.reshape(1, P),
        steps_raw.reshape(1, P),
    )


# speedup vs baseline: 1.0562x; 1.0562x over previous
import jax
import jax.numpy as jnp
from jax.experimental import pallas as pl
from jax.experimental.pallas import tpu as pltpu

_T = 1024


def _linoss_body(x_ref, br_ref, bi_ref, cr_ref, ci_ref, d_ref, ad_ref,
                 st_ref, o_ref, carry_ref, cosT_ref, sinT_ref, h0_ref,
                 p2_ref, btr_ref, bti_ref, ctr_ref, cti_ref):
    i = pl.program_id(0)
    T = x_ref.shape[0]
    P = ad_ref.shape[1]

    s = jax.nn.sigmoid(st_ref[...])
    a = jnp.maximum(ad_ref[...], 0.0)
    e2 = s * s * a
    cth = 1.0 - 0.5 * e2
    sth = jnp.sqrt(jnp.maximum(e2 * (1.0 - 0.25 * e2), 1e-24))

    @pl.when(i == 0)
    def _():
        carry_ref[...] = jnp.zeros_like(carry_ref)
        btr_ref[...] = br_ref[...].T * s
        bti_ref[...] = bi_ref[...].T * s
        ctr_ref[...] = cr_ref[...].T
        cti_ref[...] = ci_ref[...].T
        u3 = jax.lax.broadcasted_iota(jnp.int32, (T, 1), 0)
        cosT = jnp.ones((T, P), jnp.float32)
        sinT = jnp.zeros((T, P), jnp.float32)
        ck, sk = cth, sth
        d = 1
        while d < T:
            z = jnp.zeros((d, P), jnp.float32)
            shc = jnp.concatenate([z, cosT[:T - d]], axis=0)
            shs = jnp.concatenate([z, sinT[:T - d]], axis=0)
            sel = u3 >= d
            cosT = jnp.where(sel, shc * ck - shs * sk, cosT)
            sinT = jnp.where(sel, shs * ck + shc * sk, sinT)
            ck, sk = ck * ck - sk * sk, 2.0 * sk * ck
            d *= 2
        cosT_ref[...] = cosT
        sinT_ref[...] = sinT
        cot = cth / sth
        h0_ref[...] = sinT * cot + cosT
        p2_ref[...] = cosT * cot - sinT

    x = x_ref[...]
    f_r = jnp.dot(x, btr_ref[...], preferred_element_type=jnp.float32)
    f_i = jnp.dot(x, bti_ref[...], preferred_element_type=jnp.float32)

    cc = carry_ref[...]
    zm1r, zm1i = cc[0:1], cc[1:2]
    zm2r, zm2i = cc[2:3], cc[3:4]
    fpr, fpi = cc[4:5], cc[5:6]

    sm1 = s - 1.0
    fshr = jnp.concatenate([fpr, f_r[:T - 1]], axis=0)
    fshi = jnp.concatenate([fpi, f_i[:T - 1]], axis=0)
    g_r = f_r + sm1 * fshr
    g_i = f_i + sm1 * fshi

    cosT = cosT_ref[...]
    sinT = sinT_ref[...]
    qcr = cosT * g_r
    qci = cosT * g_i
    qsr = sinT * g_r
    qsi = sinT * g_i

    d = 1
    while d < T:
        z = jnp.zeros((d, P), jnp.float32)
        qcr = qcr + jnp.concatenate([z, qcr[:T - d]], axis=0)
        qci = qci + jnp.concatenate([z, qci[:T - d]], axis=0)
        qsr = qsr + jnp.concatenate([z, qsr[:T - d]], axis=0)
        qsi = qsi + jnp.concatenate([z, qsi[:T - d]], axis=0)
        d *= 2

    h0 = h0_ref[...]
    p2 = p2_ref[...]
    k1r = zm1r * cth - zm2r
    k1i = zm1i * cth - zm2i
    k2r = zm1r * sth
    k2i = zm1i * sth
    z_r = h0 * (qcr + k1r) - p2 * (qsr - k2r)
    z_i = h0 * (qci + k1i) - p2 * (qsi - k2i)

    carry_ref[0:1] = z_r[T - 1:T]
    carry_ref[1:2] = z_i[T - 1:T]
    carry_ref[2:3] = z_r[T - 2:T - 1]
    carry_ref[3:4] = z_i[T - 2:T - 1]
    carry_ref[4:5] = f_r[T - 1:T]
    carry_ref[5:6] = f_i[T - 1:T]

    o = (jnp.dot(z_r, ctr_ref[...], preferred_element_type=jnp.float32)
         - jnp.dot(z_i, cti_ref[...], preferred_element_type=jnp.float32)
         + x * d_ref[...])
    o_ref[...] = o


def kernel(input_sequence, A_diag_raw, B_real, B_img, C_real, C_img, D,
           steps_raw):
    L, H = input_sequence.shape
    P = A_diag_raw.shape[0]
    n_chunks = L // _T

    return pl.pallas_call(
        _linoss_body,
        out_shape=jax.ShapeDtypeStruct((L, H), jnp.float32),
        grid=(n_chunks,),
        in_specs=[
            pl.BlockSpec((_T, H), lambda i: (i, 0)),
            pl.BlockSpec((P, H), lambda i: (0, 0)),
            pl.BlockSpec((P, H), lambda i: (0, 0)),
            pl.BlockSpec((H, P), lambda i: (0, 0)),
            pl.BlockSpec((H, P), lambda i: (0, 0)),
            pl.BlockSpec((1, H), lambda i: (0, 0)),
            pl.BlockSpec((1, P), lambda i: (0, 0)),
            pl.BlockSpec((1, P), lambda i: (0, 0)),
        ],
        out_specs=pl.BlockSpec((_T, H), lambda i: (i, 0)),
        scratch_shapes=[
            pltpu.VMEM((8, P), jnp.float32),
            pltpu.VMEM((_T, P), jnp.float32),
            pltpu.VMEM((_T, P), jnp.float32),
            pltpu.VMEM((_T, P), jnp.float32),
            pltpu.VMEM((_T, P), jnp.float32),
            pltpu.VMEM((H, P), jnp.float32),
            pltpu.VMEM((H, P), jnp.float32),
            pltpu.VMEM((P, H), jnp.float32),
            pltpu.VMEM((P, H), jnp.float32),
        ],
        compiler_params=pltpu.CompilerParams(
            dimension_semantics=("arbitrary",),
        ),
        name="linoss_scan",
    )(
        input_sequence,
        B_real, B_img,
        C_real, C_img,
        D.reshape(1, H),
        A_diag_raw.reshape(1, P),
        steps_raw.reshape(1, P),
    )
